# Initial kernel scaffold; baseline (speedup 1.0000x reference)
import jax
import jax.numpy as jnp
from jax.experimental import pallas as pl


def kernel(predictions, labels):
    raise NotImplementedError("write your pallas kernel here")


# speedup vs baseline: 23.3273x; 23.3273x over previous
import jax
import jax.numpy as jnp
from jax.experimental import pallas as pl

_BIG = 1e9


def _boundary_mask(m):
    z_row = jnp.zeros((1, 64), jnp.float32)
    z_col = jnp.zeros((64, 1), jnp.float32)
    new = m
    new = new + jnp.concatenate([m[1:, :], z_row], axis=0)
    new = new + jnp.concatenate([z_row, m[:-1, :]], axis=0)
    new = new + jnp.concatenate([m[:, 1:], z_col], axis=1)
    new = new + jnp.concatenate([z_col, m[:, :-1]], axis=1)
    return jnp.where(m * (5.0 - new) > 0.0, 1.0, 0.0)


def _minplus_pass(pen):
    t_idx = jax.lax.broadcasted_iota(jnp.int32, (64, 1), 0).astype(jnp.float32)
    out = jnp.full(pen.shape, 4.0 * _BIG, jnp.float32)
    for s in range(64):
        d2 = (t_idx - float(s)) ** 2
        out = jnp.minimum(out, pen[s:s + 1, :] + d2)
    return out


def _transpose_blocks(x, eye):
    outs = []
    for m in range(16):
        blk = x[:, m * 64:(m + 1) * 64]
        outs.append(
            jax.lax.dot_general(blk, eye, (((0,), (0,)), ((), ())),
                                preferred_element_type=jnp.float32))
    return jnp.concatenate(outs, axis=1)


def _mhd_body(pred_ref, lab_ref, hd_ref, fail_ref):
    pred = pred_ref[...]
    lab = lab_ref[...].astype(jnp.float32)

    p0, p1, p2 = pred[:, 0], pred[:, 1], pred[:, 2]
    a_cls = [
        None,
        jnp.where((p1 > p0) & (p1 >= p2), 1.0, 0.0),
        jnp.where((p2 > p0) & (p2 > p1), 1.0, 0.0),
    ]

    edt_masks = [None] * 16
    w_masks = [None] * 16
    count_a = [None] * 8
    count_b = [None] * 8
    for jidx in range(2):
        for i in range(4):
            p = jidx * 4 + i
            a = a_cls[jidx + 1][i]
            b = lab[i, jidx + 1]
            edt_masks[p] = _boundary_mask(b)
            edt_masks[8 + p] = _boundary_mask(a)
            w_masks[p] = a * (1.0 - b)
            w_masks[8 + p] = b * (1.0 - a)
            count_a[p] = jnp.sum(a)
            count_b[p] = jnp.sum(b)

    pen = jnp.concatenate([( 1.0 - m) * _BIG for m in edt_masks], axis=1)
    w2d = jnp.concatenate(w_masks, axis=1)

    r64 = jax.lax.broadcasted_iota(jnp.int32, (64, 64), 0)
    c64 = jax.lax.broadcasted_iota(jnp.int32, (64, 64), 1)
    eye = (r64 == c64).astype(jnp.float32)

    g = _minplus_pass(pen)
    gt = _transpose_blocks(g, eye)
    d2t = _minplus_pass(gt)
    wt = _transpose_blocks(w2d, eye)

    prod = jnp.sqrt(jnp.maximum(d2t, 0.0)) * wt

    s_sum = [None] * 16
    n_edt = [None] * 16
    n_w = [None] * 16
    for m in range(16):
        blk = slice(m * 64, (m + 1) * 64)
        s_sum[m] = jnp.sum(prod[:, blk])
        n_edt[m] = jnp.sum(pen[:, blk] < 1.0)
        n_w[m] = jnp.sum(w2d[:, blk])

    hd = [None] * 8
    fail = [None] * 8
    for p in range(8):
        hd_f = jnp.where((n_w[p] > 0) & (n_edt[p] > 0),
                         s_sum[p] / jnp.maximum(count_a[p], 1.0), 0.0)
        hd_b = jnp.where((n_w[8 + p] > 0) & (n_edt[8 + p] > 0),
                         s_sum[8 + p] / jnp.maximum(count_b[p], 1.0), 0.0)
        h = jnp.maximum(hd_f, hd_b)
        hd[p] = jnp.where(count_a[p] > 0, h, 32.0)
        fail[p] = jnp.where(count_a[p] > 0, 0.0, 1.0)

    f1 = fail[0] + fail[1] + fail[2] + fail[3]
    f2 = fail[4] + fail[5] + fail[6] + fail[7]

    rr = jax.lax.broadcasted_iota(jnp.int32, (8, 128), 0)
    cc = jax.lax.broadcasted_iota(jnp.int32, (8, 128), 1)
    hdpad = jnp.zeros((8, 128), jnp.float32)
    for i in range(4):
        h1 = hd[i]
        h2 = hd[4 + i]
        vals = [(1, h1), (2, h2), (3, (h1 + h2) / 3.0), (4, h1 / 2.0)]
        for col, val in vals:
            hdpad = hdpad + jnp.where((rr == i) & (cc == col), val, 0.0)
    hd_ref[...] = hdpad

    fvals = [(1, f1), (2, f2), (3, (f1 + f2) / 3.0), (4, (f1 + f2) / 2.0)]
    fpad = jnp.zeros((8, 128), jnp.float32)
    for col, val in fvals:
        fpad = fpad + jnp.where((rr == 0) & (cc == col), val, 0.0)
    fail_ref[...] = fpad


def kernel(predictions, labels):
    hdpad, fpad = pl.pallas_call(
        _mhd_body,
        out_shape=[
            jax.ShapeDtypeStruct((8, 128), jnp.float32),
            jax.ShapeDtypeStruct((8, 128), jnp.float32),
        ],
    )(predictions, labels)
    return hdpad[:4, :5], fpad[0, :5]
